# Initial kernel scaffold; baseline (speedup 1.0000x reference)
import jax
import jax.numpy as jnp
from jax.experimental import pallas as pl


def kernel(x, embed_weight):
    raise NotImplementedError("write your pallas kernel here")


# speedup vs baseline: 2.1636x; 2.1636x over previous
import jax
import jax.numpy as jnp
from jax.experimental import pallas as pl


def _body(x_ref, w_ref, o_ref):
    y = x_ref[...] + w_ref[...][None]
    o_ref[0] = y
    o_ref[1] = y


def kernel(x, embed_weight):
    B, S, D = x.shape
    TS = 256
    out = pl.pallas_call(
        _body,
        grid=(S // TS,),
        in_specs=[
            pl.BlockSpec((B, TS, D), lambda s: (0, s, 0)),
            pl.BlockSpec((TS, D), lambda s: (s, 0)),
        ],
        out_specs=pl.BlockSpec((B, B, TS, D), lambda s: (0, 0, s, 0)),
        out_shape=jax.ShapeDtypeStruct((B, B, S, D), x.dtype),
    )(x, embed_weight)
    return out


# speedup vs baseline: 2.3156x; 1.0702x over previous
import jax
import jax.numpy as jnp
from jax.experimental import pallas as pl


def _body(x_ref, w_ref, o_ref):
    y = x_ref[...] + w_ref[...][None]
    o_ref[0] = y
    o_ref[1] = y


def kernel(x, embed_weight):
    B, S, D = x.shape
    TS = 512
    out = pl.pallas_call(
        _body,
        grid=(S // TS,),
        in_specs=[
            pl.BlockSpec((B, TS, D), lambda s: (0, s, 0)),
            pl.BlockSpec((TS, D), lambda s: (s, 0)),
        ],
        out_specs=pl.BlockSpec((B, B, TS, D), lambda s: (0, 0, s, 0)),
        out_shape=jax.ShapeDtypeStruct((B, B, S, D), x.dtype),
    )(x, embed_weight)
    return out
